# Initial kernel scaffold; baseline (speedup 1.0000x reference)
import jax
import jax.numpy as jnp
from jax.experimental import pallas as pl


def kernel(x_nchw, w1, b1, w2, b2, w3, b3, w4, b4):
    raise NotImplementedError("write your pallas kernel here")


# speedup vs baseline: 15.7276x; 15.7276x over previous
import functools

import jax
import jax.numpy as jnp
from jax.experimental import pallas as pl
from jax.experimental.pallas import tpu as pltpu


def _pool2x2(y):
    ho, wo, c = y.shape
    hq, wq = ho // 2, wo // 2
    y = y[:2 * hq, :2 * wq, :].reshape(hq, 2, wq, 2, c)
    return jnp.max(y, axis=(1, 3))


def _l1_kernel(rows_ref, w_ref, b_ref, o_ref, *, kh):
    rows = rows_ref[0]
    hp, wo, kwc = rows.shape
    ho = hp - kh + 1
    a = jnp.concatenate([rows[i:i + ho] for i in range(kh)], axis=-1)
    a = a.reshape(ho * wo, kh * kwc)
    acc = jnp.dot(a, w_ref[...], preferred_element_type=jnp.float32)
    acc = jnp.maximum(acc + b_ref[...], 0.0)
    cout = o_ref.shape[-1]
    o_ref[0] = _pool2x2(acc.reshape(ho, wo, cout)).astype(o_ref.dtype)


def _conv_kernel(x_ref, w_ref, b_ref, o_ref, *, kh, kw, pad, pool):
    x = x_ref[0]
    h, w, cin = x.shape
    x = jnp.pad(x, ((pad, pad), (pad, pad), (0, 0)))
    hp, wp = h + 2 * pad, w + 2 * pad
    ho, wo = hp - kh + 1, wp - kw + 1
    kwc = kw * cin
    rows = jnp.concatenate(
        [x[:, j:j + wo, :] for j in range(kw)], axis=-1)
    acc = None
    for i in range(kh):
        a_i = rows[i:i + ho].reshape(ho * wo, kwc)
        w_i = w_ref[i * kwc:(i + 1) * kwc, :]
        p = jnp.dot(a_i, w_i, preferred_element_type=jnp.float32)
        acc = p if acc is None else acc + p
    acc = jnp.maximum(acc + b_ref[...], 0.0)
    cout = o_ref.shape[-1]
    y = acc.reshape(ho, wo, cout)
    if pool:
        y = _pool2x2(y)
    o_ref[0] = y.astype(o_ref.dtype)


def _call_conv(body, x, in_block, out_shape, wmat, b2):
    n = x.shape[0]
    k, cout = wmat.shape
    return pl.pallas_call(
        body,
        out_shape=jax.ShapeDtypeStruct(out_shape, jnp.bfloat16),
        grid_spec=pltpu.PrefetchScalarGridSpec(
            num_scalar_prefetch=0,
            grid=(n,),
            in_specs=[
                pl.BlockSpec((1,) + in_block, lambda i: (i, 0, 0, 0)),
                pl.BlockSpec((k, cout), lambda i: (0, 0)),
                pl.BlockSpec((1, cout), lambda i: (0, 0)),
            ],
            out_specs=pl.BlockSpec((1,) + out_shape[1:], lambda i: (i, 0, 0, 0)),
        ),
        compiler_params=pltpu.CompilerParams(
            dimension_semantics=("parallel",),
            vmem_limit_bytes=48 * 1024 * 1024,
        ),
    )(x, wmat, b2)


def _wmat(w):
    cout, cin, kh, kw = w.shape
    return jnp.transpose(w, (2, 3, 1, 0)).reshape(kh * kw * cin, cout).astype(
        jnp.bfloat16)


def _conv_layer(x, w, b, pool):
    n, h, wd, _ = x.shape
    cout, _, kh, kw = w.shape
    ho, wo = h + 2 - kh + 1, wd + 2 - kw + 1
    out_shape = (n, ho // 2, wo // 2, cout) if pool else (n, ho, wo, cout)
    body = functools.partial(_conv_kernel, kh=kh, kw=kw, pad=1, pool=pool)
    return _call_conv(body, x, x.shape[1:], out_shape, _wmat(w),
                      b.reshape(1, cout).astype(jnp.float32))


@jax.jit
def kernel(x_nchw, w1, b1, w2, b2, w3, b3, w4, b4):
    n = x_nchw.shape[0]
    x = jnp.transpose(x_nchw, (0, 2, 3, 1)).astype(jnp.bfloat16)
    kh1, kw1 = w1.shape[2], w1.shape[3]
    xp = jnp.pad(x, ((0, 0), (1, 1), (1, 1), (0, 0)))
    wo1 = xp.shape[2] - kw1 + 1
    rows1 = jnp.concatenate(
        [xp[:, :, j:j + wo1, :] for j in range(kw1)], axis=-1)

    x = _call_conv(
        functools.partial(_l1_kernel, kh=kh1),
        rows1, rows1.shape[1:],
        (n, (xp.shape[1] - kh1 + 1) // 2, wo1 // 2, w1.shape[0]),
        _wmat(w1), b1.reshape(1, -1).astype(jnp.float32))

    x = _conv_layer(x, w2, b2, pool=True)
    x = _conv_layer(x, w3, b3, pool=True)
    x = _conv_layer(x, w4, b4, pool=False)

    x = jnp.transpose(x, (0, 3, 1, 2))
    return x.reshape(n, -1).astype(jnp.float32)


# speedup vs baseline: 19.4218x; 1.2349x over previous
import jax
import jax.numpy as jnp
from jax.experimental import pallas as pl
from jax.experimental.pallas import tpu as pltpu


def _l1_kernel(rows_ref, w_ref, b_ref, o_ref):
    rows = rows_ref[0]
    hp, ap, _ = rows.shape
    kh = w_ref.shape[0]
    ho = hp - kh + 1
    rowsf = rows.reshape(hp * ap, rows.shape[-1])
    acc = None
    for i in range(kh):
        a_i = rowsf[i * ap:i * ap + ho * ap]
        p = jnp.dot(a_i, w_ref[i], preferred_element_type=jnp.float32)
        acc = p if acc is None else acc + p
    acc = jnp.maximum(acc + b_ref[...], 0.0)
    m = jnp.maximum(acc[:, :64], acc[:, 64:])
    y = m.reshape(ho // 2, 2, ap, 64)
    y = jnp.maximum(y[:, 0], y[:, 1])
    o_ref[0] = y[:, :49, :].astype(o_ref.dtype)


def _make_conv_kernel(kh, kw, wpad_r, wop, pool):
    def body(x_ref, w_ref, b_ref, o_ref):
        x = x_ref[0]
        h = x.shape[0]
        x = jnp.pad(x, ((1, 1), (1, wpad_r), (0, 0)))
        hp = h + 2
        ho = hp - kh + 1
        rows = jnp.concatenate(
            [x[:, j:j + wop, :] for j in range(kw)], axis=-1)
        rowsf = rows.reshape(hp * wop, rows.shape[-1])
        acc = None
        for i in range(kh):
            a_i = rowsf[i * wop:i * wop + ho * wop]
            p = jnp.dot(a_i, w_ref[i], preferred_element_type=jnp.float32)
            acc = p if acc is None else acc + p
        acc = jnp.maximum(acc + b_ref[...], 0.0)
        cout = o_ref.shape[-1]
        y = acc.reshape(ho, wop, cout)
        if pool:
            wq = o_ref.shape[2]
            y = y[:2 * (ho // 2)].reshape(ho // 2, 2, wop, cout)
            y = jnp.maximum(y[:, 0], y[:, 1])
            y = y[:, :2 * wq, :].reshape(ho // 2, wq, 2, cout)
            y = jnp.maximum(y[:, :, 0], y[:, :, 1])
        else:
            y = y[:, :o_ref.shape[2], :]
        o_ref[0] = y.astype(o_ref.dtype)
    return body


def _call(body, x, wmat, b2, out_shape):
    n = x.shape[0]
    return pl.pallas_call(
        body,
        out_shape=jax.ShapeDtypeStruct(out_shape, jnp.bfloat16),
        grid=(n,),
        in_specs=[
            pl.BlockSpec((1,) + x.shape[1:], lambda i: (i, 0, 0, 0)),
            pl.BlockSpec(wmat.shape, lambda i: (0, 0, 0)),
            pl.BlockSpec(b2.shape, lambda i: (0, 0)),
        ],
        out_specs=pl.BlockSpec((1,) + out_shape[1:], lambda i: (i, 0, 0, 0)),
        compiler_params=pltpu.CompilerParams(
            dimension_semantics=("parallel",),
            vmem_limit_bytes=40 * 1024 * 1024,
        ),
    )(x, wmat, b2)


def _wmat3(w):
    cout, cin, kh, kw = w.shape
    return jnp.transpose(w, (2, 3, 1, 0)).reshape(kh, kw * cin, cout).astype(
        jnp.bfloat16)


def _conv_layer(x, w, b, wpad_r, wop, pool, out_hw):
    cout = w.shape[0]
    body = _make_conv_kernel(w.shape[2], w.shape[3], wpad_r, wop, pool)
    out_shape = (x.shape[0],) + out_hw + (cout,)
    return _call(body, x, _wmat3(w), b.reshape(1, cout).astype(jnp.float32),
                 out_shape)


@jax.jit
def kernel(x_nchw, w1, b1, w2, b2, w3, b3, w4, b4):
    n = x_nchw.shape[0]
    x = jnp.transpose(x_nchw, (0, 2, 3, 1)).astype(jnp.bfloat16)

    kh1, kw1 = w1.shape[2], w1.shape[3]
    xp = jnp.pad(x, ((0, 0), (1, 1), (1, 15), (0, 0)))
    taps = [xp[:, :, j:j + 112:2, :] for j in range(kw1)]
    taps += [xp[:, :, j + 1:j + 113:2, :] for j in range(kw1)]
    rows = jnp.concatenate(taps, axis=-1)

    w1m = _wmat3(w1)
    w1d = jnp.zeros((kh1, 2 * kw1 * 3, 128), jnp.bfloat16)
    w1d = w1d.at[:, :kw1 * 3, :64].set(w1m).at[:, kw1 * 3:, 64:].set(w1m)
    b1d = jnp.concatenate([b1, b1]).reshape(1, 128).astype(jnp.float32)

    x = _call(_l1_kernel, rows, w1d, b1d, (n, 49, 49, 64))
    x = _conv_layer(x, w2, b2, wpad_r=4, wop=48, pool=True, out_hw=(22, 22))
    x = _conv_layer(x, w3, b3, wpad_r=4, wop=24, pool=True, out_hw=(10, 10))
    x = _conv_layer(x, w4, b4, wpad_r=8, wop=16, pool=False, out_hw=(9, 9))

    x = jnp.transpose(x, (0, 3, 1, 2))
    return x.reshape(n, -1).astype(jnp.float32)
